# baseline (speedup 1.0000x reference)
import functools

import jax
import jax.numpy as jnp
from jax import lax
from jax.experimental import pallas as pl
from jax.experimental.pallas import tpu as pltpu
from jax.experimental.pallas import tpu_sc as plsc

NC = 2
NS = 16
NW = NC * NS
B = 4 * 8192
D = 128
BPW = B // NW
CHUNK = 128
NCHUNK = BPW // CHUNK
NBUF = 6
WPR = 8192 // BPW


def _make_gather():
    mesh = plsc.VectorSubcoreMesh(core_axis_name="c", subcore_axis_name="s")

    @functools.partial(
        pl.kernel,
        mesh=mesh,
        out_type=jax.ShapeDtypeStruct((B, D), jnp.float32),
        scratch_types=[
            pltpu.VMEM((BPW,), jnp.int32),
            pltpu.VMEM((NBUF, CHUNK, D), jnp.float32),
            pltpu.SemaphoreType.DMA,
            pltpu.SemaphoreType.DMA,
        ],
    )
    def gather_kernel(idx_hbm, table_hbm, out_hbm, idx_v, rows_v, gsem, wsem):
        wid = lax.axis_index("s") * NC + lax.axis_index("c")
        base = wid * BPW
        pltpu.sync_copy(
            idx_hbm.at[wid // WPR, pl.ds((wid % WPR) * BPW, BPW)], idx_v)
        gathers = [None] * NCHUNK
        writes = [None] * NCHUNK

        def start_gather(j):
            return pltpu.async_copy(
                table_hbm.at[idx_v.at[pl.ds(j * CHUNK, CHUNK)]],
                rows_v.at[j % NBUF], gsem)

        for j in range(NBUF - 1):
            gathers[j] = start_gather(j)
        for j in range(NCHUNK):
            gathers[j].wait()
            writes[j] = pltpu.async_copy(
                rows_v.at[j % NBUF], out_hbm.at[pl.ds(base + j * CHUNK, CHUNK)], wsem)
            nxt = j + NBUF - 1
            if nxt < NCHUNK:
                if nxt - NBUF >= 0:
                    writes[nxt - NBUF].wait()
                gathers[nxt] = start_gather(nxt)
        for j in range(NCHUNK - NBUF, NCHUNK):
            if j >= 0 and writes[j] is not None:
                writes[j].wait()

    return gather_kernel


_gather = _make_gather()


def kernel(position_ids, embed_table):
    idx = position_ids.astype(jnp.int32)
    out = _gather(idx, embed_table)
    return out.reshape(position_ids.shape + (D,))


# speedup vs baseline: 1.0121x; 1.0121x over previous
import functools

import jax
import jax.numpy as jnp
from jax import lax
from jax.experimental import pallas as pl
from jax.experimental.pallas import tpu as pltpu
from jax.experimental.pallas import tpu_sc as plsc

NC = 2
NS = 16
NW = NC * NS
B = 4 * 8192
D = 128
BPW = B // NW
CHUNK = 256
NCHUNK = BPW // CHUNK
NBUF = 3
WPR = 8192 // BPW


def _make_gather():
    mesh = plsc.VectorSubcoreMesh(core_axis_name="c", subcore_axis_name="s")

    @functools.partial(
        pl.kernel,
        mesh=mesh,
        out_type=jax.ShapeDtypeStruct((B, D), jnp.float32),
        scratch_types=[
            pltpu.VMEM((BPW,), jnp.int32),
            pltpu.VMEM((NBUF, CHUNK, D), jnp.float32),
            pltpu.SemaphoreType.DMA,
            pltpu.SemaphoreType.DMA,
        ],
    )
    def gather_kernel(idx_hbm, table_hbm, out_hbm, idx_v, rows_v, gsem, wsem):
        wid = lax.axis_index("s") * NC + lax.axis_index("c")
        base = wid * BPW
        pltpu.sync_copy(
            idx_hbm.at[wid // WPR, pl.ds((wid % WPR) * BPW, BPW)], idx_v)
        gathers = [None] * NCHUNK
        writes = [None] * NCHUNK

        def start_gather(j):
            return pltpu.async_copy(
                table_hbm.at[idx_v.at[pl.ds(j * CHUNK, CHUNK)]],
                rows_v.at[j % NBUF], gsem)

        for j in range(NBUF - 1):
            gathers[j] = start_gather(j)
        for j in range(NCHUNK):
            gathers[j].wait()
            writes[j] = pltpu.async_copy(
                rows_v.at[j % NBUF], out_hbm.at[pl.ds(base + j * CHUNK, CHUNK)], wsem)
            nxt = j + NBUF - 1
            if nxt < NCHUNK:
                if nxt - NBUF >= 0:
                    writes[nxt - NBUF].wait()
                gathers[nxt] = start_gather(nxt)
        for j in range(NCHUNK - NBUF, NCHUNK):
            if j >= 0 and writes[j] is not None:
                writes[j].wait()

    return gather_kernel


_gather = _make_gather()


def kernel(position_ids, embed_table):
    idx = position_ids.astype(jnp.int32)
    out = _gather(idx, embed_table)
    return out.reshape(position_ids.shape + (D,))


# speedup vs baseline: 1.0146x; 1.0025x over previous
import functools

import jax
import jax.numpy as jnp
from jax import lax
from jax.experimental import pallas as pl
from jax.experimental.pallas import tpu as pltpu
from jax.experimental.pallas import tpu_sc as plsc

NC = 2
NS = 16
NW = NC * NS
B = 4 * 8192
D = 128
BPW = B // NW
CHUNK = 256
NCHUNK = BPW // CHUNK
NBUF = 3
WPR = 8192 // BPW


def _make_gather():
    mesh = plsc.VectorSubcoreMesh(core_axis_name="c", subcore_axis_name="s")

    @functools.partial(
        pl.kernel,
        mesh=mesh,
        out_type=jax.ShapeDtypeStruct((B, D), jnp.float32),
        scratch_types=[
            pltpu.VMEM((BPW,), jnp.int32),
            pltpu.VMEM((NBUF, CHUNK, D), jnp.float32),
            pltpu.SemaphoreType.DMA,
            pltpu.SemaphoreType.DMA,
        ],
    )
    def gather_kernel(idx_hbm, table_hbm, out_hbm, idx_v, rows_v, gsem, wsem):
        wid = lax.axis_index("s") * NC + lax.axis_index("c")
        base = wid * BPW
        pltpu.sync_copy(
            idx_hbm.at[wid // WPR, pl.ds((wid % WPR) * BPW, BPW)], idx_v)
        gathers = [None] * NCHUNK
        writes = [None] * NCHUNK

        def start_gather(j):
            return pltpu.async_copy(
                table_hbm.at[idx_v.at[pl.ds(j * CHUNK, CHUNK)]],
                rows_v.at[j % NBUF], gsem)

        for j in range(NBUF - 1):
            gathers[j] = start_gather(j)
        for j in range(NCHUNK):
            gathers[j].wait()
            writes[j] = pltpu.async_copy(
                rows_v.at[j % NBUF], out_hbm.at[pl.ds(base + j * CHUNK, CHUNK)], wsem)
            nxt = j + NBUF - 1
            if nxt < NCHUNK:
                if nxt - NBUF >= 0:
                    writes[nxt - NBUF].wait()
                gathers[nxt] = start_gather(nxt)
        for j in range(NCHUNK - NBUF, NCHUNK):
            if j >= 0 and writes[j] is not None:
                writes[j].wait()

    return gather_kernel


_gather = _make_gather()


def kernel(position_ids, embed_table):
    idx = position_ids
    if idx.dtype != jnp.int32:
        idx = idx.astype(jnp.int32)
    out = _gather(idx, embed_table)
    return out.reshape(position_ids.shape + (D,))
